# baseline (device time: 15152 ns/iter reference)
import jax
import jax.numpy as jnp
from jax import lax
from jax.experimental import pallas as pl
from jax.experimental.pallas import tpu as pltpu

TM = 128
XS = 2
YS = 4
ZS = 4
SPLIT = XS * YS

_PLANE = tuple(
    (dx, dz) for dx in range(XS) for dz in range(ZS) if (dx, dz) != (0, 0)
)
_YLINE = (1, 2, 3)


def kernel(x, dy, gamma):
    del gamma
    m, d = x.shape
    m_loc = m // SPLIT
    n_tiles = m_loc // TM

    def body(
        x_hbm, dy_hbm, out_ref, xb, dyb, accum, sbuf,
        recvp, recvy, psems_s, psems_r, ysems_s, ysems_r, csems,
    ):
        my_x = lax.axis_index("x")
        my_y = lax.axis_index("y")
        my_z = lax.axis_index("z")

        barrier = pltpu.get_barrier_semaphore()
        for dx, dz in _PLANE:
            pl.semaphore_signal(
                barrier,
                inc=1,
                device_id=((my_x + dx) % XS, my_y, (my_z + dz) % ZS),
                device_id_type=pl.DeviceIdType.MESH,
            )
        for dyy in _YLINE:
            pl.semaphore_signal(
                barrier,
                inc=1,
                device_id=(my_x, (my_y + dyy) % YS, my_z),
                device_id_type=pl.DeviceIdType.MESH,
            )

        row0 = (my_x * YS + my_y) * m_loc
        copies = []
        for t in range(n_tiles):
            cx = pltpu.make_async_copy(
                x_hbm.at[pl.ds(row0 + t * TM, TM), :], xb.at[t], csems.at[2 * t]
            )
            cd = pltpu.make_async_copy(
                dy_hbm.at[pl.ds(row0 + t * TM, TM), :],
                dyb.at[t],
                csems.at[2 * t + 1],
            )
            cx.start()
            cd.start()
            copies.append((cx, cd))

        total = jnp.zeros((2, d), jnp.float32)
        for t in range(n_tiles):
            cx, cd = copies[t]
            cx.wait()
            cd.wait()
            xt = xb[t]
            dyt = dyb[t]
            mu = jnp.mean(xt, axis=1, keepdims=True)
            xc = xt - mu
            var = jnp.mean(xc * xc, axis=1, keepdims=True)
            rstd = lax.rsqrt(var + 1e-5)
            xhat = xc * rstd
            dg = jnp.sum(dyt * xhat, axis=0, keepdims=True)
            db = jnp.sum(dyt, axis=0, keepdims=True)
            total = total + jnp.concatenate([dg, db], axis=0)
        accum[...] = total

        pl.semaphore_wait(barrier, len(_PLANE) + len(_YLINE))

        rdmas = []
        for s, (dx, dz) in enumerate(_PLANE):
            rdma = pltpu.make_async_remote_copy(
                src_ref=accum,
                dst_ref=recvp.at[s],
                send_sem=psems_s.at[s],
                recv_sem=psems_r.at[s],
                device_id=((my_x + dx) % XS, my_y, (my_z + dz) % ZS),
                device_id_type=pl.DeviceIdType.MESH,
            )
            rdma.start()
            rdmas.append(rdma)
        for s, rdma in enumerate(rdmas):
            rdma.wait_send()
            rdma.wait_recv()
            total = total + recvp[s]
        sbuf[...] = total

        rdmas = []
        for s, dyy in enumerate(_YLINE):
            rdma = pltpu.make_async_remote_copy(
                src_ref=sbuf,
                dst_ref=recvy.at[s],
                send_sem=ysems_s.at[s],
                recv_sem=ysems_r.at[s],
                device_id=(my_x, (my_y + dyy) % YS, my_z),
                device_id_type=pl.DeviceIdType.MESH,
            )
            rdma.start()
            rdmas.append(rdma)
        for s, rdma in enumerate(rdmas):
            rdma.wait_send()
            rdma.wait_recv()
            total = total + recvy[s]
        out_ref[...] = total

    return pl.pallas_call(
        body,
        in_specs=[
            pl.BlockSpec(memory_space=pltpu.MemorySpace.HBM),
            pl.BlockSpec(memory_space=pltpu.MemorySpace.HBM),
        ],
        out_specs=pl.BlockSpec(memory_space=pltpu.MemorySpace.VMEM),
        out_shape=jax.ShapeDtypeStruct((2, d), jnp.float32),
        scratch_shapes=[
            pltpu.VMEM((n_tiles, TM, d), jnp.float32),
            pltpu.VMEM((n_tiles, TM, d), jnp.float32),
            pltpu.VMEM((2, d), jnp.float32),
            pltpu.VMEM((2, d), jnp.float32),
            pltpu.VMEM((len(_PLANE), 2, d), jnp.float32),
            pltpu.VMEM((len(_YLINE), 2, d), jnp.float32),
            pltpu.SemaphoreType.DMA((len(_PLANE),)),
            pltpu.SemaphoreType.DMA((len(_PLANE),)),
            pltpu.SemaphoreType.DMA((len(_YLINE),)),
            pltpu.SemaphoreType.DMA((len(_YLINE),)),
            pltpu.SemaphoreType.DMA((2 * n_tiles,)),
        ],
        compiler_params=pltpu.CompilerParams(collective_id=0),
    )(
        pltpu.with_memory_space_constraint(x, pltpu.MemorySpace.HBM),
        pltpu.with_memory_space_constraint(dy, pltpu.MemorySpace.HBM),
    )


# device time: 14322 ns/iter; 1.0580x vs baseline; 1.0580x over previous
import jax
import jax.numpy as jnp
from jax import lax
from jax.experimental import pallas as pl
from jax.experimental.pallas import tpu as pltpu

TM = 256
XS = 2
ZS = 4

_OFFSETS = tuple(
    (dx, dz) for dx in range(XS) for dz in range(ZS) if (dx, dz) != (0, 0)
)


def kernel(x, dy, gamma):
    del gamma
    m, d = x.shape
    m_loc = m // XS
    n_tiles = m_loc // TM
    n_peers = len(_OFFSETS)

    def body(x_hbm, dy_hbm, out_ref, xb, dyb, accum, recvs, ssems, rsems, csems):
        my_x = lax.axis_index("x")
        my_y = lax.axis_index("y")
        my_z = lax.axis_index("z")

        barrier = pltpu.get_barrier_semaphore()
        for dx, dz in _OFFSETS:
            pl.semaphore_signal(
                barrier,
                inc=1,
                device_id=((my_x + dx) % XS, my_y, (my_z + dz) % ZS),
                device_id_type=pl.DeviceIdType.MESH,
            )

        row0 = my_x * m_loc
        copies = []
        for t in range(n_tiles):
            cx = pltpu.make_async_copy(
                x_hbm.at[pl.ds(row0 + t * TM, TM), :], xb.at[t], csems.at[2 * t]
            )
            cd = pltpu.make_async_copy(
                dy_hbm.at[pl.ds(row0 + t * TM, TM), :],
                dyb.at[t],
                csems.at[2 * t + 1],
            )
            cx.start()
            cd.start()
            copies.append((cx, cd))

        total = jnp.zeros((2, d), jnp.float32)
        for t in range(n_tiles):
            cx, cd = copies[t]
            cx.wait()
            cd.wait()
            xt = xb[t]
            dyt = dyb[t]
            mu = jnp.mean(xt, axis=1, keepdims=True)
            xc = xt - mu
            var = jnp.mean(xc * xc, axis=1, keepdims=True)
            rstd = lax.rsqrt(var + 1e-5)
            xhat = xc * rstd
            dg = jnp.sum(dyt * xhat, axis=0, keepdims=True)
            db = jnp.sum(dyt, axis=0, keepdims=True)
            total = total + jnp.concatenate([dg, db], axis=0)
        accum[...] = total.astype(jnp.bfloat16)

        pl.semaphore_wait(barrier, n_peers)
        rdmas = []
        for s, (dx, dz) in enumerate(_OFFSETS):
            rdma = pltpu.make_async_remote_copy(
                src_ref=accum,
                dst_ref=recvs.at[s],
                send_sem=ssems.at[s],
                recv_sem=rsems.at[s],
                device_id=((my_x + dx) % XS, my_y, (my_z + dz) % ZS),
                device_id_type=pl.DeviceIdType.MESH,
            )
            rdma.start()
            rdmas.append(rdma)
        for s, rdma in enumerate(rdmas):
            rdma.wait_send()
            rdma.wait_recv()
            total = total + recvs[s].astype(jnp.float32)
        out_ref[...] = total

    return pl.pallas_call(
        body,
        in_specs=[
            pl.BlockSpec(memory_space=pltpu.MemorySpace.HBM),
            pl.BlockSpec(memory_space=pltpu.MemorySpace.HBM),
        ],
        out_specs=pl.BlockSpec(memory_space=pltpu.MemorySpace.VMEM),
        out_shape=jax.ShapeDtypeStruct((2, d), jnp.float32),
        scratch_shapes=[
            pltpu.VMEM((n_tiles, TM, d), jnp.float32),
            pltpu.VMEM((n_tiles, TM, d), jnp.float32),
            pltpu.VMEM((2, d), jnp.bfloat16),
            pltpu.VMEM((n_peers, 2, d), jnp.bfloat16),
            pltpu.SemaphoreType.DMA((n_peers,)),
            pltpu.SemaphoreType.DMA((n_peers,)),
            pltpu.SemaphoreType.DMA((2 * n_tiles,)),
        ],
        compiler_params=pltpu.CompilerParams(collective_id=0),
    )(
        pltpu.with_memory_space_constraint(x, pltpu.MemorySpace.HBM),
        pltpu.with_memory_space_constraint(dy, pltpu.MemorySpace.HBM),
    )
